# baseline (speedup 1.0000x reference)
import functools

import jax
import jax.numpy as jnp
from jax import lax
from jax.experimental import pallas as pl
from jax.experimental.pallas import tpu as pltpu
from jax.experimental.pallas import tpu_sc as plsc

_NUM_MODEL = 1024
_MAX_LEN = 8192

_NC = 2
_NS = 16
_NW = _NC * _NS

_MESH = plsc.VectorSubcoreMesh(core_axis_name="c", subcore_axis_name="s")

_CHUNK_LENS = (64, 56, 64, 56, 16)
_CHUNK_OFFS = (0, 64, 120, 184, 240)
_BUF_LENS = (64, 56)


@functools.partial(
    pl.kernel,
    out_type=jax.ShapeDtypeStruct((_MAX_LEN, _NUM_MODEL), jnp.float32),
    mesh=_MESH,
    scratch_types=[
        pltpu.VMEM((_MAX_LEN // _NW,), jnp.int32),
        pltpu.VMEM((_BUF_LENS[0], _NUM_MODEL), jnp.float32),
        pltpu.VMEM((_BUF_LENS[1], _NUM_MODEL), jnp.float32),
        pltpu.SemaphoreType.DMA,
        pltpu.SemaphoreType.DMA,
        pltpu.SemaphoreType.DMA,
        pltpu.SemaphoreType.DMA,
    ],
)
def _pos_embed_gather(table_hbm, pos_hbm, out_hbm, idx_v, buf_a, buf_b,
                      gsem0, gsem1, ssem0, ssem1):
    b_per_w = _MAX_LEN // _NW
    wid = lax.axis_index("s") * _NC + lax.axis_index("c")
    base = wid * b_per_w
    pltpu.sync_copy(pos_hbm.at[pl.ds(base, b_per_w)], idx_v)

    bufs = (buf_a, buf_b)
    gsems = (gsem0, gsem1)
    ssems = (ssem0, ssem1)
    nchunk = len(_CHUNK_LENS)

    def gather(c):
        b = c % 2
        ln = _CHUNK_LENS[c]
        return pltpu.async_copy(
            table_hbm.at[idx_v.at[pl.ds(_CHUNK_OFFS[c], ln)]],
            bufs[b].at[pl.ds(0, ln)], gsems[b])

    def scatter(c):
        b = c % 2
        ln = _CHUNK_LENS[c]
        return pltpu.async_copy(
            bufs[b].at[pl.ds(0, ln)],
            out_hbm.at[pl.ds(base + _CHUNK_OFFS[c], ln)], ssems[b])

    gathers = [None] * nchunk
    scatters = [None] * nchunk
    gathers[0] = gather(0)
    gathers[1] = gather(1)
    for c in range(nchunk):
        if c >= 2:
            scatters[c - 2].wait()
            gathers[c] = gather(c)
        gathers[c].wait()
        scatters[c] = scatter(c)
    scatters[nchunk - 2].wait()
    scatters[nchunk - 1].wait()


def kernel(x, pos_embed_weight, position):
    seq_len = x.shape[1]
    pos = position.reshape(-1)[:seq_len].astype(jnp.int32)
    out = _pos_embed_gather(pos_embed_weight, pos)
    return out[None]
